# baseline (speedup 1.0000x reference)
import functools

import jax
import jax.numpy as jnp
from jax import lax
from jax.experimental import pallas as pl
from jax.experimental.pallas import tpu as pltpu
from jax.experimental.pallas import tpu_sc as plsc

N = 10000
E = 160000
D = 256
HD = 128
K = 3
NEG = 0.2

CH = 128
NCHUNK = E // CH
NCORE = 2
NSUB = 16
L = 16

_VMESH = plsc.VectorSubcoreMesh(core_axis_name="c", subcore_axis_name="s")

_TSLICES = ((0, 128), (128, 128), (256, 128), (384, 128), (512, 112))

import dataclasses as _dc
_SC_CP = pltpu.CompilerParams()
if "needs_layout_passes" in pltpu.CompilerParams.__dataclass_fields__:
    _SC_CP = _dc.replace(_SC_CP, needs_layout_passes=False)


def _leaky(v):
    return jnp.where(v >= 0, v, NEG * v)


def _fc_body(x_ref, wt_ref, al_ref, ar_ref, feat_ref, el_ref, er_ref):
    x = x_ref[...]
    f = jnp.dot(x, wt_ref[...], preferred_element_type=jnp.float32)
    feat_ref[...] = f
    el_ref[...] = f @ al_ref[...]
    er_ref[...] = f @ ar_ref[...]


def _fc_stage(x, wt, al_col, ar_col):
    B = 2000
    return pl.pallas_call(
        _fc_body,
        grid=(N // B,),
        in_specs=[
            pl.BlockSpec((B, D), lambda i: (i, 0)),
            pl.BlockSpec((D, D), lambda i: (0, 0)),
            pl.BlockSpec((D, 128), lambda i: (0, 0)),
            pl.BlockSpec((D, 128), lambda i: (0, 0)),
        ],
        out_specs=[
            pl.BlockSpec((B, D), lambda i: (i, 0)),
            pl.BlockSpec((B, 128), lambda i: (i, 0)),
            pl.BlockSpec((B, 128), lambda i: (i, 0)),
        ],
        out_shape=[
            jax.ShapeDtypeStruct((N, D), jnp.float32),
            jax.ShapeDtypeStruct((N, 128), jnp.float32),
            jax.ShapeDtypeStruct((N, 128), jnp.float32),
        ],
    )(x, wt, al_col, ar_col)


def _edge_body(el_h, er_h, src_h, dst_h, b_h, ee_h, den_h,
               el_v, er_v, b_v, src_v, dst_v, ee_v, zero_v, den_sh):
    c = lax.axis_index("c")
    s = lax.axis_index("s")
    w = c * NSUB + s

    pltpu.sync_copy(el_h, el_v)
    pltpu.sync_copy(er_h, er_v)
    pltpu.sync_copy(b_h, b_v)

    @pl.loop(0, 64)
    def _(i):
        zero_v[pl.ds(i * L, L)] = jnp.zeros((L,), jnp.float32)

    @pl.when(s < 10)
    def _():
        pltpu.sync_copy(zero_v.at[pl.ds(0, 1000)], den_sh.at[pl.ds(s * 1000, 1000)])

    plsc.subcore_barrier()

    bvec = b_v[...]

    @pl.loop(w, NCHUNK, step=NCORE * NSUB)
    def _(chunk):
        base = chunk * CH
        pltpu.sync_copy(src_h.at[pl.ds(base, CH)], src_v)
        pltpu.sync_copy(dst_h.at[pl.ds(base, CH)], dst_v)
        for j in range(CH // L):
            sl = pl.ds(j * L, L)
            s16 = src_v[sl]
            d16 = dst_v[sl]
            e = plsc.load_gather(el_v, [s16]) + plsc.load_gather(er_v, [d16])
            e = jnp.where(e >= 0, e, NEG * e)
            ee_v[sl] = jnp.exp(e - bvec)
        pltpu.sync_copy(ee_v, ee_h.at[pl.ds(base, CH)])
        pltpu.sync_copy(ee_v, den_sh.at[dst_v], add=True)

    plsc.subcore_barrier()

    @pl.when(s < 10)
    def _():
        pltpu.sync_copy(den_sh.at[pl.ds(s * 1000, 1000)], zero_v.at[pl.ds(0, 1000)])
        pltpu.sync_copy(zero_v.at[pl.ds(0, 1000)],
                        den_h.at[pl.ds(c * N + s * 1000, 1000)])


def _edge_stage(el, er, src, dst, b_arr):
    f = pl.kernel(
        _edge_body,
        out_type=[
            jax.ShapeDtypeStruct((E,), jnp.float32),
            jax.ShapeDtypeStruct((NCORE * N,), jnp.float32),
        ],
        mesh=_VMESH,
        compiler_params=_SC_CP,
        scratch_types=[
            pltpu.VMEM((N,), jnp.float32),
            pltpu.VMEM((N,), jnp.float32),
            pltpu.VMEM((L,), jnp.float32),
            pltpu.VMEM((CH,), jnp.int32),
            pltpu.VMEM((CH,), jnp.int32),
            pltpu.VMEM((CH,), jnp.float32),
            pltpu.VMEM((1024,), jnp.float32),
            pltpu.VMEM_SHARED((N,), jnp.float32),
        ],
    )
    return f(el, er, src, dst, b_arr)


def _norm_body(den_h, dst_h, ee_h, a_h, d0_v, d1_v, dst_v, ee_v, a_v):
    c = lax.axis_index("c")
    s = lax.axis_index("s")
    w = c * NSUB + s

    pltpu.sync_copy(den_h.at[pl.ds(0, N)], d0_v)
    pltpu.sync_copy(den_h.at[pl.ds(N, N)], d1_v)

    @pl.loop(0, N // L)
    def _(i):
        sl = pl.ds(i * L, L)
        d0_v[sl] = d0_v[sl] + d1_v[sl]

    @pl.loop(w, NCHUNK, step=NCORE * NSUB)
    def _(chunk):
        base = chunk * CH
        pltpu.sync_copy(dst_h.at[pl.ds(base, CH)], dst_v)
        pltpu.sync_copy(ee_h.at[pl.ds(base, CH)], ee_v)
        for j in range(CH // L):
            sl = pl.ds(j * L, L)
            d16 = dst_v[sl]
            a_v[sl] = ee_v[sl] / plsc.load_gather(d0_v, [d16])
        pltpu.sync_copy(a_v, a_h.at[pl.ds(base, CH)])


def _norm_stage(den2, dst, ee):
    f = pl.kernel(
        _norm_body,
        out_type=jax.ShapeDtypeStruct((E,), jnp.float32),
        mesh=_VMESH,
        compiler_params=_SC_CP,
        scratch_types=[
            pltpu.VMEM((N,), jnp.float32),
            pltpu.VMEM((N,), jnp.float32),
            pltpu.VMEM((CH,), jnp.int32),
            pltpu.VMEM((CH,), jnp.float32),
            pltpu.VMEM((CH,), jnp.float32),
        ],
    )
    return f(den2, dst, ee)


HCH = 80
HNCH = E // HCH
HNT = HNCH // NSUB


def _hop_body(h2_h, gsrc_h, dst_h, a_h, hn2_h,
              gi0, gi1, gi2, gi3, di0, di1, di2, di3, ai0, ai1, ai2, ai3,
              g0, g1, o0, o1,
              is0, is1, is2, is3, gsem0, gsem1, ssem0, ssem1, acc_sh):
    c = lax.axis_index("c")
    s = lax.axis_index("s")

    gis = (gi0, gi1, gi2, gi3)
    dis = (di0, di1, di2, di3)
    ais = (ai0, ai1, ai2, ai3)
    isems = (is0, is1, is2, is3)
    gbufs = (g0, g1)
    obufs = (o0, o1)
    gsems = (gsem0, gsem1)
    ssems = (ssem0, ssem1)

    @pl.loop(0, HCH)
    def _(i):
        row = g0.at[i]
        for j in range(HD // L):
            row[pl.ds(j * L, L)] = jnp.zeros((L,), jnp.float32)

    for i in range(7):
        pltpu.sync_copy(g0.at[pl.ds(0, HCH)],
                        acc_sh.at[pl.ds(s * 624 + i * HCH, HCH)])
    pltpu.sync_copy(g0.at[pl.ds(0, 64)], acc_sh.at[pl.ds(s * 624 + 560, 64)])

    @pl.when(s == NSUB - 1)
    def _():
        pltpu.sync_copy(g0.at[pl.ds(0, 16)], acc_sh.at[pl.ds(9984, 16)])

    plsc.subcore_barrier()

    base_chunk = s * HNT

    def cbase(j):
        return (base_chunk + j) * HCH

    def issue_idx(j, b):
        pltpu.async_copy(gsrc_h.at[pl.ds(c * E + cbase(j), HCH)], gis[b], isems[b])
        pltpu.async_copy(dst_h.at[pl.ds(cbase(j), HCH)], dis[b], isems[b])
        pltpu.async_copy(a_h.at[pl.ds(cbase(j), HCH)], ais[b], isems[b])

    def wait_idx(j, b):
        pltpu.make_async_copy(gsrc_h.at[pl.ds(c * E + cbase(j), HCH)], gis[b], isems[b]).wait()
        pltpu.make_async_copy(dst_h.at[pl.ds(cbase(j), HCH)], dis[b], isems[b]).wait()
        pltpu.make_async_copy(a_h.at[pl.ds(cbase(j), HCH)], ais[b], isems[b]).wait()

    def issue_gather(b4, b2):
        pltpu.async_copy(h2_h.at[gis[b4]], gbufs[b2], gsems[b2])

    def wait_gather(b4, b2):
        pltpu.make_async_copy(h2_h.at[gis[b4]], gbufs[b2], gsems[b2]).wait()

    def do_scale(b4, b2):
        arow = ais[b4]
        g = gbufs[b2]
        o = obufs[b2]

        @pl.loop(0, HCH)
        def _(i):
            bc = plsc.load_gather(arow, [jnp.full((L,), i, jnp.int32)])
            rg = g.at[i]
            ro = o.at[i]
            for f in range(HD // L):
                slf = pl.ds(f * L, L)
                ro[slf] = rg[slf] * bc

    def issue_scatter(b4, b2):
        pltpu.async_copy(obufs[b2], acc_sh.at[dis[b4]], ssems[b2], add=True)

    def wait_scatter(b4, b2):
        pltpu.make_async_copy(obufs[b2], acc_sh.at[dis[b4]], ssems[b2]).wait()

    def step(j, b4, b2, wait_sc, issue_i):
        wait_idx(j + 1, (b4 + 1) % 4)
        issue_gather((b4 + 1) % 4, (b2 + 1) % 2)
        wait_gather(b4, b2)
        if wait_sc:
            wait_scatter((b4 + 2) % 4, b2)
        do_scale(b4, b2)
        issue_scatter(b4, b2)
        if issue_i:
            issue_idx(j + 2, (b4 + 2) % 4)

    issue_idx(0, 0)
    issue_idx(1, 1)
    issue_idx(2, 2)
    wait_idx(0, 0)
    issue_gather(0, 0)
    wait_idx(1, 1)
    issue_gather(1, 1)
    wait_gather(0, 0)
    do_scale(0, 0)
    issue_scatter(0, 0)
    wait_idx(2, 2)
    issue_gather(2, 0)
    wait_gather(1, 1)
    do_scale(1, 1)
    issue_scatter(1, 1)
    issue_idx(3, 3)

    @pl.loop(0, (HNT - 5) // 4)
    def _(g):
        for r in range(4):
            step(2 + g * 4 + r, (2 + r) % 4, r % 2, True, True)

    step(HNT - 3, (HNT - 3) % 4, (HNT - 3) % 2, True, True)
    wait_idx(HNT - 1, (HNT - 1) % 4)
    issue_gather((HNT - 1) % 4, (HNT - 1) % 2)
    wait_gather((HNT - 2) % 4, (HNT - 2) % 2)
    wait_scatter((HNT - 4) % 4, (HNT - 4) % 2)
    do_scale((HNT - 2) % 4, (HNT - 2) % 2)
    issue_scatter((HNT - 2) % 4, (HNT - 2) % 2)
    wait_gather((HNT - 1) % 4, (HNT - 1) % 2)
    wait_scatter((HNT - 3) % 4, (HNT - 3) % 2)
    do_scale((HNT - 1) % 4, (HNT - 1) % 2)
    issue_scatter((HNT - 1) % 4, (HNT - 1) % 2)
    wait_scatter((HNT - 2) % 4, (HNT - 2) % 2)
    wait_scatter((HNT - 1) % 4, (HNT - 1) % 2)

    plsc.subcore_barrier()

    coff = c * N
    for i in range(7):
        row0 = s * 624 + i * HCH
        pltpu.sync_copy(acc_sh.at[pl.ds(row0, HCH)], g0.at[pl.ds(0, HCH)])
        pltpu.sync_copy(g0.at[pl.ds(0, HCH)], hn2_h.at[pl.ds(coff + row0, HCH)])
    row0 = s * 624 + 560
    pltpu.sync_copy(acc_sh.at[pl.ds(row0, 64)], g0.at[pl.ds(0, 64)])
    pltpu.sync_copy(g0.at[pl.ds(0, 64)], hn2_h.at[pl.ds(coff + row0, 64)])

    @pl.when(s == NSUB - 1)
    def _():
        pltpu.sync_copy(acc_sh.at[pl.ds(9984, 16)], g0.at[pl.ds(0, 16)])
        pltpu.sync_copy(g0.at[pl.ds(0, 16)], hn2_h.at[pl.ds(coff + 9984, 16)])


def _hop_stage(h2, gsrc, dst, a):
    f = pl.kernel(
        _hop_body,
        out_type=jax.ShapeDtypeStruct((NCORE * N, HD), jnp.float32),
        mesh=_VMESH,
        compiler_params=_SC_CP,
        scratch_types=(
            [pltpu.VMEM((HCH,), jnp.int32) for _ in range(4)]
            + [pltpu.VMEM((HCH,), jnp.int32) for _ in range(4)]
            + [pltpu.VMEM((HCH,), jnp.float32) for _ in range(4)]
            + [pltpu.VMEM((HCH, HD), jnp.float32) for _ in range(4)]
            + [pltpu.SemaphoreType.DMA for _ in range(8)]
            + [pltpu.VMEM_SHARED((N, HD), jnp.float32)]
        ),
    )
    return f(h2, gsrc, dst, a)


def _combine_body(h0a, h0b, h1a, h1b, h2a, h2b, h3a, h3b, p_ref, out_ref):
    p = p_ref[...]
    ha = [h0a[...], h1a[...], h2a[...], h3a[...]]
    hb = [h0b[...], h1b[...], h2b[...], h3b[...]]
    for k in range(K + 1):
        ha[k] = ha[k] + p[k:k + 1, :HD]
        hb[k] = hb[k] + p[k:k + 1, HD:]
    hal_a, hal_b = p[4:5, :HD], p[4:5, HD:]
    har_a, har_b = p[5:6, :HD], p[5:6, HD:]
    a_r = jnp.sum(ha[0] * har_a, axis=1, keepdims=True) + \
        jnp.sum(hb[0] * har_b, axis=1, keepdims=True)
    logits = [jnp.sum(ha[k] * hal_a, axis=1, keepdims=True) +
              jnp.sum(hb[k] * hal_b, axis=1, keepdims=True) + a_r
              for k in range(K + 1)]
    logits = [_leaky(lg) for lg in logits]
    mx = functools.reduce(jnp.maximum, logits)
    exps = [jnp.exp(lg - mx) for lg in logits]
    den = functools.reduce(jnp.add, exps)
    outa = functools.reduce(jnp.add, [ha[k] * (exps[k] / den) for k in range(K + 1)])
    outb = functools.reduce(jnp.add, [hb[k] * (exps[k] / den) for k in range(K + 1)])
    out_ref[:, :HD] = outa + p[6:7, :HD]
    out_ref[:, HD:] = outb + p[6:7, HD:]


def _combine_stage(hs2, params):
    B = 2000
    NB = N // B
    ins = []
    specs = []
    for h2 in hs2:
        ins.append(h2)
        specs.append(pl.BlockSpec((B, HD), lambda i: (i, 0)))
        ins.append(h2)
        specs.append(pl.BlockSpec((B, HD), lambda i: (NB + i, 0)))
    ins.append(params)
    specs.append(pl.BlockSpec((8, D), lambda i: (0, 0)))
    return pl.pallas_call(
        _combine_body,
        grid=(NB,),
        in_specs=specs,
        out_specs=pl.BlockSpec((B, D), lambda i: (i, 0)),
        out_shape=jax.ShapeDtypeStruct((N, D), jnp.float32),
    )(*ins)


def kernel(x, edge_index, W_fc, attn_l, attn_r, hop_attn_l, hop_attn_r, position_emb, bias):
    src = edge_index[0]
    dst = edge_index[1]
    al_col = jnp.broadcast_to(attn_l.reshape(D, 1), (D, 128))
    ar_col = jnp.broadcast_to(attn_r.reshape(D, 1), (D, 128))
    feat, el_w, er_w = _fc_stage(x, W_fc.T, al_col, ar_col)
    el = el_w[:, 0]
    er = er_w[:, 0]

    b_const = jnp.max(el) + jnp.max(er)
    b_arr = jnp.full((L,), b_const, jnp.float32)

    ee, den2 = _edge_stage(el, er, src, dst, b_arr)
    a = _norm_stage(den2, dst, ee)

    gsrc = jnp.concatenate([src, src + N])
    h2 = jnp.concatenate([feat[:, :HD], feat[:, HD:]], axis=0)
    hs2 = [h2]
    for _ in range(K):
        h2 = _hop_stage(h2, gsrc, dst, a)
        hs2.append(h2)

    pe = position_emb.reshape(K + 1, D)
    params = jnp.concatenate([
        pe,
        hop_attn_l.reshape(1, D),
        hop_attn_r.reshape(1, D),
        bias.reshape(1, D),
        jnp.zeros((1, D), jnp.float32),
    ], axis=0)
    rst = _combine_stage(hs2, params)
    return rst.reshape(N, 1, D)


# speedup vs baseline: 2.3938x; 2.3938x over previous
import functools

import jax
import jax.numpy as jnp
from jax import lax
from jax.experimental import pallas as pl
from jax.experimental.pallas import tpu as pltpu
from jax.experimental.pallas import tpu_sc as plsc

N = 10000
E = 160000
D = 256
HD = 128
K = 3
NEG = 0.2

CH = 128
NCHUNK = E // CH
NCORE = 2
NSUB = 16
L = 16

_VMESH = plsc.VectorSubcoreMesh(core_axis_name="c", subcore_axis_name="s")

_TSLICES = ((0, 128), (128, 128), (256, 128), (384, 128), (512, 112))

import dataclasses as _dc
_SC_CP = pltpu.CompilerParams()
if "needs_layout_passes" in pltpu.CompilerParams.__dataclass_fields__:
    _SC_CP = _dc.replace(_SC_CP, needs_layout_passes=False)


def _leaky(v):
    return jnp.where(v >= 0, v, NEG * v)


def _fc_body(x_ref, wt_ref, al_ref, ar_ref, feat_ref, el_ref, er_ref):
    x = x_ref[...]
    f = jnp.dot(x, wt_ref[...], preferred_element_type=jnp.float32)
    feat_ref[...] = f
    el_ref[...] = f @ al_ref[...]
    er_ref[...] = f @ ar_ref[...]


def _fc_stage(x, wt, al_col, ar_col):
    B = 2000
    return pl.pallas_call(
        _fc_body,
        grid=(N // B,),
        in_specs=[
            pl.BlockSpec((B, D), lambda i: (i, 0)),
            pl.BlockSpec((D, D), lambda i: (0, 0)),
            pl.BlockSpec((D, 128), lambda i: (0, 0)),
            pl.BlockSpec((D, 128), lambda i: (0, 0)),
        ],
        out_specs=[
            pl.BlockSpec((B, D), lambda i: (i, 0)),
            pl.BlockSpec((B, 128), lambda i: (i, 0)),
            pl.BlockSpec((B, 128), lambda i: (i, 0)),
        ],
        out_shape=[
            jax.ShapeDtypeStruct((N, D), jnp.float32),
            jax.ShapeDtypeStruct((N, 128), jnp.float32),
            jax.ShapeDtypeStruct((N, 128), jnp.float32),
        ],
    )(x, wt, al_col, ar_col)


def _edge_body(el_h, er_h, src_h, dst_h, b_h, ee_h, den_h,
               el_v, er_v, b_v, src_v, dst_v, ee_v, zero_v, den_sh):
    c = lax.axis_index("c")
    s = lax.axis_index("s")
    w = c * NSUB + s

    pltpu.sync_copy(el_h, el_v)
    pltpu.sync_copy(er_h, er_v)
    pltpu.sync_copy(b_h, b_v)

    @pl.loop(0, 64)
    def _(i):
        zero_v[pl.ds(i * L, L)] = jnp.zeros((L,), jnp.float32)

    @pl.when(s < 10)
    def _():
        pltpu.sync_copy(zero_v.at[pl.ds(0, 1000)], den_sh.at[pl.ds(s * 1000, 1000)])

    plsc.subcore_barrier()

    bvec = b_v[...]

    @pl.loop(w, NCHUNK, step=NCORE * NSUB)
    def _(chunk):
        base = chunk * CH
        pltpu.sync_copy(src_h.at[pl.ds(base, CH)], src_v)
        pltpu.sync_copy(dst_h.at[pl.ds(base, CH)], dst_v)
        for j in range(CH // L):
            sl = pl.ds(j * L, L)
            s16 = src_v[sl]
            d16 = dst_v[sl]
            e = plsc.load_gather(el_v, [s16]) + plsc.load_gather(er_v, [d16])
            e = jnp.where(e >= 0, e, NEG * e)
            ee_v[sl] = jnp.exp(e - bvec)
        pltpu.sync_copy(ee_v, ee_h.at[pl.ds(base, CH)])
        pltpu.sync_copy(ee_v, den_sh.at[dst_v], add=True)

    plsc.subcore_barrier()

    @pl.when(s < 10)
    def _():
        pltpu.sync_copy(den_sh.at[pl.ds(s * 1000, 1000)], zero_v.at[pl.ds(0, 1000)])
        pltpu.sync_copy(zero_v.at[pl.ds(0, 1000)],
                        den_h.at[pl.ds(c * N + s * 1000, 1000)])


def _edge_stage(el, er, src, dst, b_arr):
    f = pl.kernel(
        _edge_body,
        out_type=[
            jax.ShapeDtypeStruct((E,), jnp.float32),
            jax.ShapeDtypeStruct((NCORE * N,), jnp.float32),
        ],
        mesh=_VMESH,
        compiler_params=_SC_CP,
        scratch_types=[
            pltpu.VMEM((N,), jnp.float32),
            pltpu.VMEM((N,), jnp.float32),
            pltpu.VMEM((L,), jnp.float32),
            pltpu.VMEM((CH,), jnp.int32),
            pltpu.VMEM((CH,), jnp.int32),
            pltpu.VMEM((CH,), jnp.float32),
            pltpu.VMEM((1024,), jnp.float32),
            pltpu.VMEM_SHARED((N,), jnp.float32),
        ],
    )
    return f(el, er, src, dst, b_arr)


def _norm_body(den_h, dst_h, ee_h, a_h, d0_v, d1_v, dst_v, ee_v, a_v):
    c = lax.axis_index("c")
    s = lax.axis_index("s")
    w = c * NSUB + s

    pltpu.sync_copy(den_h.at[pl.ds(0, N)], d0_v)
    pltpu.sync_copy(den_h.at[pl.ds(N, N)], d1_v)

    @pl.loop(0, N // L)
    def _(i):
        sl = pl.ds(i * L, L)
        d0_v[sl] = d0_v[sl] + d1_v[sl]

    @pl.loop(w, NCHUNK, step=NCORE * NSUB)
    def _(chunk):
        base = chunk * CH
        pltpu.sync_copy(dst_h.at[pl.ds(base, CH)], dst_v)
        pltpu.sync_copy(ee_h.at[pl.ds(base, CH)], ee_v)
        for j in range(CH // L):
            sl = pl.ds(j * L, L)
            d16 = dst_v[sl]
            a_v[sl] = ee_v[sl] / plsc.load_gather(d0_v, [d16])
        pltpu.sync_copy(a_v, a_h.at[pl.ds(base, CH)])


def _norm_stage(den2, dst, ee):
    f = pl.kernel(
        _norm_body,
        out_type=jax.ShapeDtypeStruct((E,), jnp.float32),
        mesh=_VMESH,
        compiler_params=_SC_CP,
        scratch_types=[
            pltpu.VMEM((N,), jnp.float32),
            pltpu.VMEM((N,), jnp.float32),
            pltpu.VMEM((CH,), jnp.int32),
            pltpu.VMEM((CH,), jnp.float32),
            pltpu.VMEM((CH,), jnp.float32),
        ],
    )
    return f(den2, dst, ee)


HCH = 80
HNCH = E // HCH
HNT = HNCH // NSUB


def _hop_body(h2_h, gsrc_h, dst_h, a_h, hn2_h,
              gi0, gi1, gi2, gi3, di0, di1, di2, di3, ai0, ai1, ai2, ai3,
              g0, g1, o0, o1,
              is0, is1, is2, is3, gsem0, gsem1, ssem0, ssem1, acc_sh):
    c = lax.axis_index("c")
    s = lax.axis_index("s")

    gis = (gi0, gi1, gi2, gi3)
    dis = (di0, di1, di2, di3)
    ais = (ai0, ai1, ai2, ai3)
    isems = (is0, is1, is2, is3)
    gbufs = (g0, g1)
    obufs = (o0, o1)
    gsems = (gsem0, gsem1)
    ssems = (ssem0, ssem1)

    @pl.loop(0, HCH)
    def _(i):
        row = g0.at[i]
        for j in range(HD // L):
            row[pl.ds(j * L, L)] = jnp.zeros((L,), jnp.float32)

    for i in range(7):
        pltpu.sync_copy(g0.at[pl.ds(0, HCH)],
                        acc_sh.at[pl.ds(s * 624 + i * HCH, HCH)])
    pltpu.sync_copy(g0.at[pl.ds(0, 64)], acc_sh.at[pl.ds(s * 624 + 560, 64)])

    @pl.when(s == NSUB - 1)
    def _():
        pltpu.sync_copy(g0.at[pl.ds(0, 16)], acc_sh.at[pl.ds(9984, 16)])

    plsc.subcore_barrier()

    base_chunk = s * HNT

    def cbase(j):
        return (base_chunk + j) * HCH

    def issue_idx(j, b):
        pltpu.async_copy(gsrc_h.at[pl.ds(c * E + cbase(j), HCH)], gis[b], isems[b])
        pltpu.async_copy(dst_h.at[pl.ds(cbase(j), HCH)], dis[b], isems[b])
        pltpu.async_copy(a_h.at[pl.ds(cbase(j), HCH)], ais[b], isems[b])

    def wait_idx(j, b):
        pltpu.make_async_copy(gsrc_h.at[pl.ds(c * E + cbase(j), HCH)], gis[b], isems[b]).wait()
        pltpu.make_async_copy(dst_h.at[pl.ds(cbase(j), HCH)], dis[b], isems[b]).wait()
        pltpu.make_async_copy(a_h.at[pl.ds(cbase(j), HCH)], ais[b], isems[b]).wait()

    def issue_gather(b4, b2):
        pltpu.async_copy(h2_h.at[gis[b4]], gbufs[b2], gsems[b2])

    def wait_gather(b4, b2):
        pltpu.make_async_copy(h2_h.at[gis[b4]], gbufs[b2], gsems[b2]).wait()

    def do_scale(b4, b2):
        arow = ais[b4]
        g = gbufs[b2]
        o = obufs[b2]

        @plsc.parallel_loop(0, HCH, unroll=4)
        def _(i):
            bc = plsc.load_gather(arow, [jnp.full((L,), i, jnp.int32)])
            rg = g.at[i]
            ro = o.at[i]
            for f in range(HD // L):
                slf = pl.ds(f * L, L)
                ro[slf] = rg[slf] * bc

    def issue_scatter(b4, b2):
        pltpu.async_copy(obufs[b2], acc_sh.at[dis[b4]], ssems[b2], add=True)

    def wait_scatter(b4, b2):
        pltpu.make_async_copy(obufs[b2], acc_sh.at[dis[b4]], ssems[b2]).wait()

    def step(j, b4, b2, wait_sc, issue_i):
        wait_idx(j + 1, (b4 + 1) % 4)
        issue_gather((b4 + 1) % 4, (b2 + 1) % 2)
        wait_gather(b4, b2)
        if wait_sc:
            wait_scatter((b4 + 2) % 4, b2)
        do_scale(b4, b2)
        issue_scatter(b4, b2)
        if issue_i:
            issue_idx(j + 2, (b4 + 2) % 4)

    issue_idx(0, 0)
    issue_idx(1, 1)
    issue_idx(2, 2)
    wait_idx(0, 0)
    issue_gather(0, 0)
    wait_idx(1, 1)
    issue_gather(1, 1)
    wait_gather(0, 0)
    do_scale(0, 0)
    issue_scatter(0, 0)
    wait_idx(2, 2)
    issue_gather(2, 0)
    wait_gather(1, 1)
    do_scale(1, 1)
    issue_scatter(1, 1)
    issue_idx(3, 3)

    @pl.loop(0, (HNT - 5) // 4)
    def _(g):
        for r in range(4):
            step(2 + g * 4 + r, (2 + r) % 4, r % 2, True, True)

    step(HNT - 3, (HNT - 3) % 4, (HNT - 3) % 2, True, True)
    wait_idx(HNT - 1, (HNT - 1) % 4)
    issue_gather((HNT - 1) % 4, (HNT - 1) % 2)
    wait_gather((HNT - 2) % 4, (HNT - 2) % 2)
    wait_scatter((HNT - 4) % 4, (HNT - 4) % 2)
    do_scale((HNT - 2) % 4, (HNT - 2) % 2)
    issue_scatter((HNT - 2) % 4, (HNT - 2) % 2)
    wait_gather((HNT - 1) % 4, (HNT - 1) % 2)
    wait_scatter((HNT - 3) % 4, (HNT - 3) % 2)
    do_scale((HNT - 1) % 4, (HNT - 1) % 2)
    issue_scatter((HNT - 1) % 4, (HNT - 1) % 2)
    wait_scatter((HNT - 2) % 4, (HNT - 2) % 2)
    wait_scatter((HNT - 1) % 4, (HNT - 1) % 2)

    plsc.subcore_barrier()

    coff = c * N
    for i in range(7):
        row0 = s * 624 + i * HCH
        pltpu.sync_copy(acc_sh.at[pl.ds(row0, HCH)], g0.at[pl.ds(0, HCH)])
        pltpu.sync_copy(g0.at[pl.ds(0, HCH)], hn2_h.at[pl.ds(coff + row0, HCH)])
    row0 = s * 624 + 560
    pltpu.sync_copy(acc_sh.at[pl.ds(row0, 64)], g0.at[pl.ds(0, 64)])
    pltpu.sync_copy(g0.at[pl.ds(0, 64)], hn2_h.at[pl.ds(coff + row0, 64)])

    @pl.when(s == NSUB - 1)
    def _():
        pltpu.sync_copy(acc_sh.at[pl.ds(9984, 16)], g0.at[pl.ds(0, 16)])
        pltpu.sync_copy(g0.at[pl.ds(0, 16)], hn2_h.at[pl.ds(coff + 9984, 16)])


def _hop_stage(h2, gsrc, dst, a):
    f = pl.kernel(
        _hop_body,
        out_type=jax.ShapeDtypeStruct((NCORE * N, HD), jnp.float32),
        mesh=_VMESH,
        compiler_params=_SC_CP,
        scratch_types=(
            [pltpu.VMEM((HCH,), jnp.int32) for _ in range(4)]
            + [pltpu.VMEM((HCH,), jnp.int32) for _ in range(4)]
            + [pltpu.VMEM((HCH,), jnp.float32) for _ in range(4)]
            + [pltpu.VMEM((HCH, HD), jnp.float32) for _ in range(4)]
            + [pltpu.SemaphoreType.DMA for _ in range(8)]
            + [pltpu.VMEM_SHARED((N, HD), jnp.float32)]
        ),
    )
    return f(h2, gsrc, dst, a)


def _combine_body(h0a, h0b, h1a, h1b, h2a, h2b, h3a, h3b, p_ref, out_ref):
    p = p_ref[...]
    ha = [h0a[...], h1a[...], h2a[...], h3a[...]]
    hb = [h0b[...], h1b[...], h2b[...], h3b[...]]
    for k in range(K + 1):
        ha[k] = ha[k] + p[k:k + 1, :HD]
        hb[k] = hb[k] + p[k:k + 1, HD:]
    hal_a, hal_b = p[4:5, :HD], p[4:5, HD:]
    har_a, har_b = p[5:6, :HD], p[5:6, HD:]
    a_r = jnp.sum(ha[0] * har_a, axis=1, keepdims=True) + \
        jnp.sum(hb[0] * har_b, axis=1, keepdims=True)
    logits = [jnp.sum(ha[k] * hal_a, axis=1, keepdims=True) +
              jnp.sum(hb[k] * hal_b, axis=1, keepdims=True) + a_r
              for k in range(K + 1)]
    logits = [_leaky(lg) for lg in logits]
    mx = functools.reduce(jnp.maximum, logits)
    exps = [jnp.exp(lg - mx) for lg in logits]
    den = functools.reduce(jnp.add, exps)
    outa = functools.reduce(jnp.add, [ha[k] * (exps[k] / den) for k in range(K + 1)])
    outb = functools.reduce(jnp.add, [hb[k] * (exps[k] / den) for k in range(K + 1)])
    out_ref[:, :HD] = outa + p[6:7, :HD]
    out_ref[:, HD:] = outb + p[6:7, HD:]


def _combine_stage(hs2, params):
    B = 2000
    NB = N // B
    ins = []
    specs = []
    for h2 in hs2:
        ins.append(h2)
        specs.append(pl.BlockSpec((B, HD), lambda i: (i, 0)))
        ins.append(h2)
        specs.append(pl.BlockSpec((B, HD), lambda i: (NB + i, 0)))
    ins.append(params)
    specs.append(pl.BlockSpec((8, D), lambda i: (0, 0)))
    return pl.pallas_call(
        _combine_body,
        grid=(NB,),
        in_specs=specs,
        out_specs=pl.BlockSpec((B, D), lambda i: (i, 0)),
        out_shape=jax.ShapeDtypeStruct((N, D), jnp.float32),
    )(*ins)


def kernel(x, edge_index, W_fc, attn_l, attn_r, hop_attn_l, hop_attn_r, position_emb, bias):
    src = edge_index[0]
    dst = edge_index[1]
    al_col = jnp.broadcast_to(attn_l.reshape(D, 1), (D, 128))
    ar_col = jnp.broadcast_to(attn_r.reshape(D, 1), (D, 128))
    feat, el_w, er_w = _fc_stage(x, W_fc.T, al_col, ar_col)
    el = el_w[:, 0]
    er = er_w[:, 0]

    b_const = jnp.max(el) + jnp.max(er)
    b_arr = jnp.full((L,), b_const, jnp.float32)

    ee, den2 = _edge_stage(el, er, src, dst, b_arr)
    a = _norm_stage(den2, dst, ee)

    gsrc = jnp.concatenate([src, src + N])
    h2 = jnp.concatenate([feat[:, :HD], feat[:, HD:]], axis=0)
    hs2 = [h2]
    for _ in range(K):
        h2 = _hop_stage(h2, gsrc, dst, a)
        hs2.append(h2)

    pe = position_emb.reshape(K + 1, D)
    params = jnp.concatenate([
        pe,
        hop_attn_l.reshape(1, D),
        hop_attn_r.reshape(1, D),
        bias.reshape(1, D),
        jnp.zeros((1, D), jnp.float32),
    ], axis=0)
    rst = _combine_stage(hs2, params)
    return rst.reshape(N, 1, D)


# speedup vs baseline: 2.3987x; 1.0020x over previous
import functools

import jax
import jax.numpy as jnp
from jax import lax
from jax.experimental import pallas as pl
from jax.experimental.pallas import tpu as pltpu
from jax.experimental.pallas import tpu_sc as plsc

N = 10000
E = 160000
D = 256
HD = 128
K = 3
NEG = 0.2

CH = 128
NCHUNK = E // CH
NCORE = 2
NSUB = 16
L = 16

_VMESH = plsc.VectorSubcoreMesh(core_axis_name="c", subcore_axis_name="s")

_TSLICES = ((0, 128), (128, 128), (256, 128), (384, 128), (512, 112))

import dataclasses as _dc
_SC_CP = pltpu.CompilerParams()
if "needs_layout_passes" in pltpu.CompilerParams.__dataclass_fields__:
    _SC_CP = _dc.replace(_SC_CP, needs_layout_passes=False)


def _leaky(v):
    return jnp.where(v >= 0, v, NEG * v)


def _fc_body(x_ref, wt_ref, al_ref, ar_ref, feat_ref, el_ref, er_ref):
    x = x_ref[...]
    f = jnp.dot(x, wt_ref[...], preferred_element_type=jnp.float32)
    feat_ref[...] = f
    el_ref[...] = f @ al_ref[...]
    er_ref[...] = f @ ar_ref[...]


def _fc_stage(x, wt, al_col, ar_col):
    B = 2000
    return pl.pallas_call(
        _fc_body,
        grid=(N // B,),
        in_specs=[
            pl.BlockSpec((B, D), lambda i: (i, 0)),
            pl.BlockSpec((D, D), lambda i: (0, 0)),
            pl.BlockSpec((D, 128), lambda i: (0, 0)),
            pl.BlockSpec((D, 128), lambda i: (0, 0)),
        ],
        out_specs=[
            pl.BlockSpec((B, D), lambda i: (i, 0)),
            pl.BlockSpec((B, 128), lambda i: (i, 0)),
            pl.BlockSpec((B, 128), lambda i: (i, 0)),
        ],
        out_shape=[
            jax.ShapeDtypeStruct((N, D), jnp.float32),
            jax.ShapeDtypeStruct((N, 128), jnp.float32),
            jax.ShapeDtypeStruct((N, 128), jnp.float32),
        ],
    )(x, wt, al_col, ar_col)


def _edge_body(el_h, er_h, src_h, dst_h, b_h, ee_h, den_h,
               el_v, er_v, b_v, src_v, dst_v, ee_v, zero_v, den_sh):
    c = lax.axis_index("c")
    s = lax.axis_index("s")
    w = c * NSUB + s

    pltpu.sync_copy(el_h, el_v)
    pltpu.sync_copy(er_h, er_v)
    pltpu.sync_copy(b_h, b_v)

    @pl.loop(0, 64)
    def _(i):
        zero_v[pl.ds(i * L, L)] = jnp.zeros((L,), jnp.float32)

    @pl.when(s < 10)
    def _():
        pltpu.sync_copy(zero_v.at[pl.ds(0, 1000)], den_sh.at[pl.ds(s * 1000, 1000)])

    plsc.subcore_barrier()

    bvec = b_v[...]

    @pl.loop(w, NCHUNK, step=NCORE * NSUB)
    def _(chunk):
        base = chunk * CH
        pltpu.sync_copy(src_h.at[pl.ds(base, CH)], src_v)
        pltpu.sync_copy(dst_h.at[pl.ds(base, CH)], dst_v)
        for j in range(CH // L):
            sl = pl.ds(j * L, L)
            s16 = src_v[sl]
            d16 = dst_v[sl]
            e = plsc.load_gather(el_v, [s16]) + plsc.load_gather(er_v, [d16])
            e = jnp.where(e >= 0, e, NEG * e)
            ee_v[sl] = jnp.exp(e - bvec)
        pltpu.sync_copy(ee_v, ee_h.at[pl.ds(base, CH)])
        pltpu.sync_copy(ee_v, den_sh.at[dst_v], add=True)

    plsc.subcore_barrier()

    @pl.when(s < 10)
    def _():
        pltpu.sync_copy(den_sh.at[pl.ds(s * 1000, 1000)], zero_v.at[pl.ds(0, 1000)])
        pltpu.sync_copy(zero_v.at[pl.ds(0, 1000)],
                        den_h.at[pl.ds(c * N + s * 1000, 1000)])


def _edge_stage(el, er, src, dst, b_arr):
    f = pl.kernel(
        _edge_body,
        out_type=[
            jax.ShapeDtypeStruct((E,), jnp.float32),
            jax.ShapeDtypeStruct((NCORE * N,), jnp.float32),
        ],
        mesh=_VMESH,
        compiler_params=_SC_CP,
        scratch_types=[
            pltpu.VMEM((N,), jnp.float32),
            pltpu.VMEM((N,), jnp.float32),
            pltpu.VMEM((L,), jnp.float32),
            pltpu.VMEM((CH,), jnp.int32),
            pltpu.VMEM((CH,), jnp.int32),
            pltpu.VMEM((CH,), jnp.float32),
            pltpu.VMEM((1024,), jnp.float32),
            pltpu.VMEM_SHARED((N,), jnp.float32),
        ],
    )
    return f(el, er, src, dst, b_arr)


def _norm_body(den_h, dst_h, ee_h, a_h, d0_v, d1_v, dst_v, ee_v, a_v):
    c = lax.axis_index("c")
    s = lax.axis_index("s")
    w = c * NSUB + s

    pltpu.sync_copy(den_h.at[pl.ds(0, N)], d0_v)
    pltpu.sync_copy(den_h.at[pl.ds(N, N)], d1_v)

    @pl.loop(0, N // L)
    def _(i):
        sl = pl.ds(i * L, L)
        d0_v[sl] = d0_v[sl] + d1_v[sl]

    @pl.loop(w, NCHUNK, step=NCORE * NSUB)
    def _(chunk):
        base = chunk * CH
        pltpu.sync_copy(dst_h.at[pl.ds(base, CH)], dst_v)
        pltpu.sync_copy(ee_h.at[pl.ds(base, CH)], ee_v)
        for j in range(CH // L):
            sl = pl.ds(j * L, L)
            d16 = dst_v[sl]
            a_v[sl] = ee_v[sl] / plsc.load_gather(d0_v, [d16])
        pltpu.sync_copy(a_v, a_h.at[pl.ds(base, CH)])


def _norm_stage(den2, dst, ee):
    f = pl.kernel(
        _norm_body,
        out_type=jax.ShapeDtypeStruct((E,), jnp.float32),
        mesh=_VMESH,
        compiler_params=_SC_CP,
        scratch_types=[
            pltpu.VMEM((N,), jnp.float32),
            pltpu.VMEM((N,), jnp.float32),
            pltpu.VMEM((CH,), jnp.int32),
            pltpu.VMEM((CH,), jnp.float32),
            pltpu.VMEM((CH,), jnp.float32),
        ],
    )
    return f(den2, dst, ee)


HCH = 80
HNCH = E // HCH
HNT = HNCH // NSUB


def _hop_body(h2_h, gsrc_h, dst_h, a_h, hn2_h,
              gi0, gi1, gi2, gi3, di0, di1, di2, di3, ai0, ai1, ai2, ai3,
              g0, g1, o0, o1,
              is0, is1, is2, is3, gsem0, gsem1, ssem0, ssem1, acc_sh):
    c = lax.axis_index("c")
    s = lax.axis_index("s")

    gis = (gi0, gi1, gi2, gi3)
    dis = (di0, di1, di2, di3)
    ais = (ai0, ai1, ai2, ai3)
    isems = (is0, is1, is2, is3)
    gbufs = (g0, g1)
    obufs = (o0, o1)
    gsems = (gsem0, gsem1)
    ssems = (ssem0, ssem1)

    @pl.loop(0, HCH)
    def _(i):
        row = g0.at[i]
        for j in range(HD // L):
            row[pl.ds(j * L, L)] = jnp.zeros((L,), jnp.float32)

    for i in range(7):
        pltpu.sync_copy(g0.at[pl.ds(0, HCH)],
                        acc_sh.at[pl.ds(s * 624 + i * HCH, HCH)])
    pltpu.sync_copy(g0.at[pl.ds(0, 64)], acc_sh.at[pl.ds(s * 624 + 560, 64)])

    @pl.when(s == NSUB - 1)
    def _():
        pltpu.sync_copy(g0.at[pl.ds(0, 16)], acc_sh.at[pl.ds(9984, 16)])

    plsc.subcore_barrier()

    base_chunk = s * HNT

    def cbase(j):
        return (base_chunk + j) * HCH

    def issue_idx(j, b):
        pltpu.async_copy(gsrc_h.at[pl.ds(c * E + cbase(j), HCH)], gis[b], isems[b])
        pltpu.async_copy(dst_h.at[pl.ds(cbase(j), HCH)], dis[b], isems[b])
        pltpu.async_copy(a_h.at[pl.ds(cbase(j), HCH)], ais[b], isems[b])

    def wait_idx(j, b):
        pltpu.make_async_copy(gsrc_h.at[pl.ds(c * E + cbase(j), HCH)], gis[b], isems[b]).wait()
        pltpu.make_async_copy(dst_h.at[pl.ds(cbase(j), HCH)], dis[b], isems[b]).wait()
        pltpu.make_async_copy(a_h.at[pl.ds(cbase(j), HCH)], ais[b], isems[b]).wait()

    def issue_gather(b4, b2):
        pltpu.async_copy(h2_h.at[gis[b4]], gbufs[b2], gsems[b2])

    def wait_gather(b4, b2):
        pltpu.make_async_copy(h2_h.at[gis[b4]], gbufs[b2], gsems[b2]).wait()

    def do_scale(b4, b2):
        arow = ais[b4]
        g = gbufs[b2]
        o = obufs[b2]

        @plsc.parallel_loop(0, HCH, unroll=8)
        def _(i):
            bc = plsc.load_gather(arow, [jnp.full((L,), i, jnp.int32)])
            rg = g.at[i]
            ro = o.at[i]
            for f in range(HD // L):
                slf = pl.ds(f * L, L)
                ro[slf] = rg[slf] * bc

    def issue_scatter(b4, b2):
        pltpu.async_copy(obufs[b2], acc_sh.at[dis[b4]], ssems[b2], add=True)

    def wait_scatter(b4, b2):
        pltpu.make_async_copy(obufs[b2], acc_sh.at[dis[b4]], ssems[b2]).wait()

    def step(j, b4, b2, wait_sc, issue_i):
        wait_idx(j + 1, (b4 + 1) % 4)
        issue_gather((b4 + 1) % 4, (b2 + 1) % 2)
        wait_gather(b4, b2)
        if wait_sc:
            wait_scatter((b4 + 2) % 4, b2)
        do_scale(b4, b2)
        issue_scatter(b4, b2)
        if issue_i:
            issue_idx(j + 2, (b4 + 2) % 4)

    issue_idx(0, 0)
    issue_idx(1, 1)
    issue_idx(2, 2)
    wait_idx(0, 0)
    issue_gather(0, 0)
    wait_idx(1, 1)
    issue_gather(1, 1)
    wait_gather(0, 0)
    do_scale(0, 0)
    issue_scatter(0, 0)
    wait_idx(2, 2)
    issue_gather(2, 0)
    wait_gather(1, 1)
    do_scale(1, 1)
    issue_scatter(1, 1)
    issue_idx(3, 3)

    @pl.loop(0, (HNT - 5) // 4)
    def _(g):
        for r in range(4):
            step(2 + g * 4 + r, (2 + r) % 4, r % 2, True, True)

    step(HNT - 3, (HNT - 3) % 4, (HNT - 3) % 2, True, True)
    wait_idx(HNT - 1, (HNT - 1) % 4)
    issue_gather((HNT - 1) % 4, (HNT - 1) % 2)
    wait_gather((HNT - 2) % 4, (HNT - 2) % 2)
    wait_scatter((HNT - 4) % 4, (HNT - 4) % 2)
    do_scale((HNT - 2) % 4, (HNT - 2) % 2)
    issue_scatter((HNT - 2) % 4, (HNT - 2) % 2)
    wait_gather((HNT - 1) % 4, (HNT - 1) % 2)
    wait_scatter((HNT - 3) % 4, (HNT - 3) % 2)
    do_scale((HNT - 1) % 4, (HNT - 1) % 2)
    issue_scatter((HNT - 1) % 4, (HNT - 1) % 2)
    wait_scatter((HNT - 2) % 4, (HNT - 2) % 2)
    wait_scatter((HNT - 1) % 4, (HNT - 1) % 2)

    plsc.subcore_barrier()

    coff = c * N
    for i in range(7):
        row0 = s * 624 + i * HCH
        pltpu.sync_copy(acc_sh.at[pl.ds(row0, HCH)], g0.at[pl.ds(0, HCH)])
        pltpu.sync_copy(g0.at[pl.ds(0, HCH)], hn2_h.at[pl.ds(coff + row0, HCH)])
    row0 = s * 624 + 560
    pltpu.sync_copy(acc_sh.at[pl.ds(row0, 64)], g0.at[pl.ds(0, 64)])
    pltpu.sync_copy(g0.at[pl.ds(0, 64)], hn2_h.at[pl.ds(coff + row0, 64)])

    @pl.when(s == NSUB - 1)
    def _():
        pltpu.sync_copy(acc_sh.at[pl.ds(9984, 16)], g0.at[pl.ds(0, 16)])
        pltpu.sync_copy(g0.at[pl.ds(0, 16)], hn2_h.at[pl.ds(coff + 9984, 16)])


def _hop_stage(h2, gsrc, dst, a):
    f = pl.kernel(
        _hop_body,
        out_type=jax.ShapeDtypeStruct((NCORE * N, HD), jnp.float32),
        mesh=_VMESH,
        compiler_params=_SC_CP,
        scratch_types=(
            [pltpu.VMEM((HCH,), jnp.int32) for _ in range(4)]
            + [pltpu.VMEM((HCH,), jnp.int32) for _ in range(4)]
            + [pltpu.VMEM((HCH,), jnp.float32) for _ in range(4)]
            + [pltpu.VMEM((HCH, HD), jnp.float32) for _ in range(4)]
            + [pltpu.SemaphoreType.DMA for _ in range(8)]
            + [pltpu.VMEM_SHARED((N, HD), jnp.float32)]
        ),
    )
    return f(h2, gsrc, dst, a)


def _combine_body(h0a, h0b, h1a, h1b, h2a, h2b, h3a, h3b, p_ref, out_ref):
    p = p_ref[...]
    ha = [h0a[...], h1a[...], h2a[...], h3a[...]]
    hb = [h0b[...], h1b[...], h2b[...], h3b[...]]
    for k in range(K + 1):
        ha[k] = ha[k] + p[k:k + 1, :HD]
        hb[k] = hb[k] + p[k:k + 1, HD:]
    hal_a, hal_b = p[4:5, :HD], p[4:5, HD:]
    har_a, har_b = p[5:6, :HD], p[5:6, HD:]
    a_r = jnp.sum(ha[0] * har_a, axis=1, keepdims=True) + \
        jnp.sum(hb[0] * har_b, axis=1, keepdims=True)
    logits = [jnp.sum(ha[k] * hal_a, axis=1, keepdims=True) +
              jnp.sum(hb[k] * hal_b, axis=1, keepdims=True) + a_r
              for k in range(K + 1)]
    logits = [_leaky(lg) for lg in logits]
    mx = functools.reduce(jnp.maximum, logits)
    exps = [jnp.exp(lg - mx) for lg in logits]
    den = functools.reduce(jnp.add, exps)
    outa = functools.reduce(jnp.add, [ha[k] * (exps[k] / den) for k in range(K + 1)])
    outb = functools.reduce(jnp.add, [hb[k] * (exps[k] / den) for k in range(K + 1)])
    out_ref[:, :HD] = outa + p[6:7, :HD]
    out_ref[:, HD:] = outb + p[6:7, HD:]


def _combine_stage(hs2, params):
    B = 2000
    NB = N // B
    ins = []
    specs = []
    for h2 in hs2:
        ins.append(h2)
        specs.append(pl.BlockSpec((B, HD), lambda i: (i, 0)))
        ins.append(h2)
        specs.append(pl.BlockSpec((B, HD), lambda i: (NB + i, 0)))
    ins.append(params)
    specs.append(pl.BlockSpec((8, D), lambda i: (0, 0)))
    return pl.pallas_call(
        _combine_body,
        grid=(NB,),
        in_specs=specs,
        out_specs=pl.BlockSpec((B, D), lambda i: (i, 0)),
        out_shape=jax.ShapeDtypeStruct((N, D), jnp.float32),
    )(*ins)


def kernel(x, edge_index, W_fc, attn_l, attn_r, hop_attn_l, hop_attn_r, position_emb, bias):
    src = edge_index[0]
    dst = edge_index[1]
    al_col = jnp.broadcast_to(attn_l.reshape(D, 1), (D, 128))
    ar_col = jnp.broadcast_to(attn_r.reshape(D, 1), (D, 128))
    feat, el_w, er_w = _fc_stage(x, W_fc.T, al_col, ar_col)
    el = el_w[:, 0]
    er = er_w[:, 0]

    b_const = jnp.max(el) + jnp.max(er)
    b_arr = jnp.full((L,), b_const, jnp.float32)

    ee, den2 = _edge_stage(el, er, src, dst, b_arr)
    a = _norm_stage(den2, dst, ee)

    gsrc = jnp.concatenate([src, src + N])
    h2 = jnp.concatenate([feat[:, :HD], feat[:, HD:]], axis=0)
    hs2 = [h2]
    for _ in range(K):
        h2 = _hop_stage(h2, gsrc, dst, a)
        hs2.append(h2)

    pe = position_emb.reshape(K + 1, D)
    params = jnp.concatenate([
        pe,
        hop_attn_l.reshape(1, D),
        hop_attn_r.reshape(1, D),
        bias.reshape(1, D),
        jnp.zeros((1, D), jnp.float32),
    ], axis=0)
    rst = _combine_stage(hs2, params)
    return rst.reshape(N, 1, D)


# speedup vs baseline: 2.5293x; 1.0544x over previous
import functools

import jax
import jax.numpy as jnp
from jax import lax
from jax.experimental import pallas as pl
from jax.experimental.pallas import tpu as pltpu
from jax.experimental.pallas import tpu_sc as plsc

N = 10000
E = 160000
D = 256
HD = 128
K = 3
NEG = 0.2

CH = 128
NCHUNK = E // CH
NCORE = 2
NSUB = 16
L = 16

_VMESH = plsc.VectorSubcoreMesh(core_axis_name="c", subcore_axis_name="s")

_TSLICES = ((0, 128), (128, 128), (256, 128), (384, 128), (512, 112))

import dataclasses as _dc
_SC_CP = pltpu.CompilerParams()
if "needs_layout_passes" in pltpu.CompilerParams.__dataclass_fields__:
    _SC_CP = _dc.replace(_SC_CP, needs_layout_passes=False)


def _leaky(v):
    return jnp.where(v >= 0, v, NEG * v)


def _fc_body(x_ref, wt_ref, al_ref, ar_ref, feat_ref, el_ref, er_ref):
    x = x_ref[...]
    f = jnp.dot(x, wt_ref[...], preferred_element_type=jnp.float32)
    feat_ref[...] = f
    el_ref[...] = f @ al_ref[...]
    er_ref[...] = f @ ar_ref[...]


def _fc_stage(x, wt, al_col, ar_col):
    B = 2000
    return pl.pallas_call(
        _fc_body,
        grid=(N // B,),
        in_specs=[
            pl.BlockSpec((B, D), lambda i: (i, 0)),
            pl.BlockSpec((D, D), lambda i: (0, 0)),
            pl.BlockSpec((D, 128), lambda i: (0, 0)),
            pl.BlockSpec((D, 128), lambda i: (0, 0)),
        ],
        out_specs=[
            pl.BlockSpec((B, D), lambda i: (i, 0)),
            pl.BlockSpec((B, 128), lambda i: (i, 0)),
            pl.BlockSpec((B, 128), lambda i: (i, 0)),
        ],
        out_shape=[
            jax.ShapeDtypeStruct((N, D), jnp.float32),
            jax.ShapeDtypeStruct((N, 128), jnp.float32),
            jax.ShapeDtypeStruct((N, 128), jnp.float32),
        ],
    )(x, wt, al_col, ar_col)


def _edge_body(el_h, er_h, src_h, dst_h, b_h, ee_h, den_h,
               el_v, er_v, b_v, src_v, dst_v, ee_v, zero_v, den_sh):
    c = lax.axis_index("c")
    s = lax.axis_index("s")
    w = c * NSUB + s

    pltpu.sync_copy(el_h, el_v)
    pltpu.sync_copy(er_h, er_v)
    pltpu.sync_copy(b_h, b_v)

    @pl.loop(0, 64)
    def _(i):
        zero_v[pl.ds(i * L, L)] = jnp.zeros((L,), jnp.float32)

    @pl.when(s < 10)
    def _():
        pltpu.sync_copy(zero_v.at[pl.ds(0, 1000)], den_sh.at[pl.ds(s * 1000, 1000)])

    plsc.subcore_barrier()

    bvec = b_v[...]

    @pl.loop(w, NCHUNK, step=NCORE * NSUB)
    def _(chunk):
        base = chunk * CH
        pltpu.sync_copy(src_h.at[pl.ds(base, CH)], src_v)
        pltpu.sync_copy(dst_h.at[pl.ds(base, CH)], dst_v)
        for j in range(CH // L):
            sl = pl.ds(j * L, L)
            s16 = src_v[sl]
            d16 = dst_v[sl]
            e = plsc.load_gather(el_v, [s16]) + plsc.load_gather(er_v, [d16])
            e = jnp.where(e >= 0, e, NEG * e)
            ee_v[sl] = jnp.exp(e - bvec)
        pltpu.sync_copy(ee_v, ee_h.at[pl.ds(base, CH)])
        pltpu.sync_copy(ee_v, den_sh.at[dst_v], add=True)

    plsc.subcore_barrier()

    @pl.when(s < 10)
    def _():
        pltpu.sync_copy(den_sh.at[pl.ds(s * 1000, 1000)], zero_v.at[pl.ds(0, 1000)])
        pltpu.sync_copy(zero_v.at[pl.ds(0, 1000)],
                        den_h.at[pl.ds(c * N + s * 1000, 1000)])


def _edge_stage(el, er, src, dst, b_arr):
    f = pl.kernel(
        _edge_body,
        out_type=[
            jax.ShapeDtypeStruct((E,), jnp.float32),
            jax.ShapeDtypeStruct((NCORE * N,), jnp.float32),
        ],
        mesh=_VMESH,
        compiler_params=_SC_CP,
        scratch_types=[
            pltpu.VMEM((N,), jnp.float32),
            pltpu.VMEM((N,), jnp.float32),
            pltpu.VMEM((L,), jnp.float32),
            pltpu.VMEM((CH,), jnp.int32),
            pltpu.VMEM((CH,), jnp.int32),
            pltpu.VMEM((CH,), jnp.float32),
            pltpu.VMEM((1024,), jnp.float32),
            pltpu.VMEM_SHARED((N,), jnp.float32),
        ],
    )
    return f(el, er, src, dst, b_arr)


def _norm_body(den_h, dst_h, ee_h, a_h, d0_v, d1_v, dst_v, ee_v, a_v):
    c = lax.axis_index("c")
    s = lax.axis_index("s")
    w = c * NSUB + s

    pltpu.sync_copy(den_h.at[pl.ds(0, N)], d0_v)
    pltpu.sync_copy(den_h.at[pl.ds(N, N)], d1_v)

    @pl.loop(0, N // L)
    def _(i):
        sl = pl.ds(i * L, L)
        d0_v[sl] = d0_v[sl] + d1_v[sl]

    @pl.loop(w, NCHUNK, step=NCORE * NSUB)
    def _(chunk):
        base = chunk * CH
        pltpu.sync_copy(dst_h.at[pl.ds(base, CH)], dst_v)
        pltpu.sync_copy(ee_h.at[pl.ds(base, CH)], ee_v)
        for j in range(CH // L):
            sl = pl.ds(j * L, L)
            d16 = dst_v[sl]
            a_v[sl] = ee_v[sl] / plsc.load_gather(d0_v, [d16])
        pltpu.sync_copy(a_v, a_h.at[pl.ds(base, CH)])


def _norm_stage(den2, dst, ee):
    f = pl.kernel(
        _norm_body,
        out_type=jax.ShapeDtypeStruct((E,), jnp.float32),
        mesh=_VMESH,
        compiler_params=_SC_CP,
        scratch_types=[
            pltpu.VMEM((N,), jnp.float32),
            pltpu.VMEM((N,), jnp.float32),
            pltpu.VMEM((CH,), jnp.int32),
            pltpu.VMEM((CH,), jnp.float32),
            pltpu.VMEM((CH,), jnp.float32),
        ],
    )
    return f(den2, dst, ee)


HCH = 80
HNCH = E // HCH
HNT = HNCH // NSUB


def _hop_body(h2_h, gsrc_h, dst_h, a_h, dinv_h, hn2_h,
              gi0, gi1, gi2, gi3, di0, di1, di2, di3, ai0, ai1, ai2, ai3,
              g0, g1, o0, o1, dinv_v,
              is0, is1, is2, is3, gsem0, gsem1, ssem0, ssem1, acc_sh):
    c = lax.axis_index("c")
    s = lax.axis_index("s")

    gis = (gi0, gi1, gi2, gi3)
    dis = (di0, di1, di2, di3)
    ais = (ai0, ai1, ai2, ai3)
    isems = (is0, is1, is2, is3)
    gbufs = (g0, g1)
    obufs = (o0, o1)
    gsems = (gsem0, gsem1)
    ssems = (ssem0, ssem1)

    @pl.loop(0, HCH)
    def _(i):
        row = g0.at[i]
        for j in range(HD // L):
            row[pl.ds(j * L, L)] = jnp.zeros((L,), jnp.float32)

    for i in range(7):
        pltpu.sync_copy(g0.at[pl.ds(0, HCH)],
                        acc_sh.at[pl.ds(s * 624 + i * HCH, HCH)])
    pltpu.sync_copy(g0.at[pl.ds(0, 64)], acc_sh.at[pl.ds(s * 624 + 560, 64)])

    @pl.when(s == NSUB - 1)
    def _():
        pltpu.sync_copy(g0.at[pl.ds(0, 16)], acc_sh.at[pl.ds(9984, 16)])

    pltpu.sync_copy(dinv_h.at[pl.ds(s * 624, 640)], dinv_v)

    plsc.subcore_barrier()

    base_chunk = s * HNT

    def cbase(j):
        return (base_chunk + j) * HCH

    def issue_idx(j, b):
        pltpu.async_copy(gsrc_h.at[pl.ds(c * E + cbase(j), HCH)], gis[b], isems[b])
        pltpu.async_copy(dst_h.at[pl.ds(cbase(j), HCH)], dis[b], isems[b])
        pltpu.async_copy(a_h.at[pl.ds(cbase(j), HCH)], ais[b], isems[b])

    def wait_idx(j, b):
        pltpu.make_async_copy(gsrc_h.at[pl.ds(c * E + cbase(j), HCH)], gis[b], isems[b]).wait()
        pltpu.make_async_copy(dst_h.at[pl.ds(cbase(j), HCH)], dis[b], isems[b]).wait()
        pltpu.make_async_copy(a_h.at[pl.ds(cbase(j), HCH)], ais[b], isems[b]).wait()

    def issue_gather(b4, b2):
        pltpu.async_copy(h2_h.at[gis[b4]], gbufs[b2], gsems[b2])

    def wait_gather(b4, b2):
        pltpu.make_async_copy(h2_h.at[gis[b4]], gbufs[b2], gsems[b2]).wait()

    def do_scale(b4, b2):
        arow = ais[b4]
        g = gbufs[b2]
        o = obufs[b2]

        @plsc.parallel_loop(0, HCH, unroll=8)
        def _(i):
            bc = plsc.load_gather(arow, [jnp.full((L,), i, jnp.int32)])
            rg = g.at[i]
            ro = o.at[i]
            for f in range(HD // L):
                slf = pl.ds(f * L, L)
                ro[slf] = rg[slf] * bc

    def issue_scatter(b4, b2):
        pltpu.async_copy(obufs[b2], acc_sh.at[dis[b4]], ssems[b2], add=True)

    def wait_scatter(b4, b2):
        pltpu.make_async_copy(obufs[b2], acc_sh.at[dis[b4]], ssems[b2]).wait()

    def step(j, b4, b2, wait_sc, issue_i):
        wait_idx(j + 1, (b4 + 1) % 4)
        issue_gather((b4 + 1) % 4, (b2 + 1) % 2)
        wait_gather(b4, b2)
        if wait_sc:
            wait_scatter((b4 + 2) % 4, b2)
        do_scale(b4, b2)
        issue_scatter(b4, b2)
        if issue_i:
            issue_idx(j + 2, (b4 + 2) % 4)

    issue_idx(0, 0)
    issue_idx(1, 1)
    issue_idx(2, 2)
    wait_idx(0, 0)
    issue_gather(0, 0)
    wait_idx(1, 1)
    issue_gather(1, 1)
    wait_gather(0, 0)
    do_scale(0, 0)
    issue_scatter(0, 0)
    wait_idx(2, 2)
    issue_gather(2, 0)
    wait_gather(1, 1)
    do_scale(1, 1)
    issue_scatter(1, 1)
    issue_idx(3, 3)

    @pl.loop(0, (HNT - 5) // 4)
    def _(g):
        for r in range(4):
            step(2 + g * 4 + r, (2 + r) % 4, r % 2, True, True)

    step(HNT - 3, (HNT - 3) % 4, (HNT - 3) % 2, True, True)
    wait_idx(HNT - 1, (HNT - 1) % 4)
    issue_gather((HNT - 1) % 4, (HNT - 1) % 2)
    wait_gather((HNT - 2) % 4, (HNT - 2) % 2)
    wait_scatter((HNT - 4) % 4, (HNT - 4) % 2)
    do_scale((HNT - 2) % 4, (HNT - 2) % 2)
    issue_scatter((HNT - 2) % 4, (HNT - 2) % 2)
    wait_gather((HNT - 1) % 4, (HNT - 1) % 2)
    wait_scatter((HNT - 3) % 4, (HNT - 3) % 2)
    do_scale((HNT - 1) % 4, (HNT - 1) % 2)
    issue_scatter((HNT - 1) % 4, (HNT - 1) % 2)
    wait_scatter((HNT - 2) % 4, (HNT - 2) % 2)
    wait_scatter((HNT - 1) % 4, (HNT - 1) % 2)

    plsc.subcore_barrier()

    def scale_out(local0, nrows):
        @plsc.parallel_loop(0, nrows, unroll=4)
        def _(r):
            bc = plsc.load_gather(dinv_v, [jnp.full((L,), local0 + r, jnp.int32)])
            rg = g0.at[r]
            for f in range(HD // L):
                slf = pl.ds(f * L, L)
                rg[slf] = rg[slf] * bc

    coff = c * N
    for i in range(7):
        row0 = s * 624 + i * HCH
        pltpu.sync_copy(acc_sh.at[pl.ds(row0, HCH)], g0.at[pl.ds(0, HCH)])
        scale_out(i * HCH, HCH)
        pltpu.sync_copy(g0.at[pl.ds(0, HCH)], hn2_h.at[pl.ds(coff + row0, HCH)])
    row0 = s * 624 + 560
    pltpu.sync_copy(acc_sh.at[pl.ds(row0, 64)], g0.at[pl.ds(0, 64)])
    scale_out(560, 64)
    pltpu.sync_copy(g0.at[pl.ds(0, 64)], hn2_h.at[pl.ds(coff + row0, 64)])

    @pl.when(s == NSUB - 1)
    def _():
        pltpu.sync_copy(acc_sh.at[pl.ds(9984, 16)], g0.at[pl.ds(0, 16)])
        scale_out(624, 16)
        pltpu.sync_copy(g0.at[pl.ds(0, 16)], hn2_h.at[pl.ds(coff + 9984, 16)])


def _hop_stage(h2, gsrc, dst, ee, dinv):
    f = pl.kernel(
        _hop_body,
        out_type=jax.ShapeDtypeStruct((NCORE * N, HD), jnp.float32),
        mesh=_VMESH,
        compiler_params=_SC_CP,
        scratch_types=(
            [pltpu.VMEM((HCH,), jnp.int32) for _ in range(4)]
            + [pltpu.VMEM((HCH,), jnp.int32) for _ in range(4)]
            + [pltpu.VMEM((HCH,), jnp.float32) for _ in range(4)]
            + [pltpu.VMEM((HCH, HD), jnp.float32) for _ in range(4)]
            + [pltpu.VMEM((640,), jnp.float32)]
            + [pltpu.SemaphoreType.DMA for _ in range(8)]
            + [pltpu.VMEM_SHARED((N, HD), jnp.float32)]
        ),
    )
    return f(h2, gsrc, dst, ee, dinv)


def _combine_body(h0a, h0b, h1a, h1b, h2a, h2b, h3a, h3b, p_ref, out_ref):
    p = p_ref[...]
    ha = [h0a[...], h1a[...], h2a[...], h3a[...]]
    hb = [h0b[...], h1b[...], h2b[...], h3b[...]]
    for k in range(K + 1):
        ha[k] = ha[k] + p[k:k + 1, :HD]
        hb[k] = hb[k] + p[k:k + 1, HD:]
    hal_a, hal_b = p[4:5, :HD], p[4:5, HD:]
    har_a, har_b = p[5:6, :HD], p[5:6, HD:]
    a_r = jnp.sum(ha[0] * har_a, axis=1, keepdims=True) + \
        jnp.sum(hb[0] * har_b, axis=1, keepdims=True)
    logits = [jnp.sum(ha[k] * hal_a, axis=1, keepdims=True) +
              jnp.sum(hb[k] * hal_b, axis=1, keepdims=True) + a_r
              for k in range(K + 1)]
    logits = [_leaky(lg) for lg in logits]
    mx = functools.reduce(jnp.maximum, logits)
    exps = [jnp.exp(lg - mx) for lg in logits]
    den = functools.reduce(jnp.add, exps)
    outa = functools.reduce(jnp.add, [ha[k] * (exps[k] / den) for k in range(K + 1)])
    outb = functools.reduce(jnp.add, [hb[k] * (exps[k] / den) for k in range(K + 1)])
    out_ref[:, :HD] = outa + p[6:7, :HD]
    out_ref[:, HD:] = outb + p[6:7, HD:]


def _combine_stage(hs2, params):
    B = 2000
    NB = N // B
    ins = []
    specs = []
    for h2 in hs2:
        ins.append(h2)
        specs.append(pl.BlockSpec((B, HD), lambda i: (i, 0)))
        ins.append(h2)
        specs.append(pl.BlockSpec((B, HD), lambda i: (NB + i, 0)))
    ins.append(params)
    specs.append(pl.BlockSpec((8, D), lambda i: (0, 0)))
    return pl.pallas_call(
        _combine_body,
        grid=(NB,),
        in_specs=specs,
        out_specs=pl.BlockSpec((B, D), lambda i: (i, 0)),
        out_shape=jax.ShapeDtypeStruct((N, D), jnp.float32),
    )(*ins)


def kernel(x, edge_index, W_fc, attn_l, attn_r, hop_attn_l, hop_attn_r, position_emb, bias):
    src = edge_index[0]
    dst = edge_index[1]
    al_col = jnp.broadcast_to(attn_l.reshape(D, 1), (D, 128))
    ar_col = jnp.broadcast_to(attn_r.reshape(D, 1), (D, 128))
    feat, el_w, er_w = _fc_stage(x, W_fc.T, al_col, ar_col)
    el = el_w[:, 0]
    er = er_w[:, 0]

    b_const = jnp.max(el) + jnp.max(er)
    b_arr = jnp.full((L,), b_const, jnp.float32)

    ee, den2 = _edge_stage(el, er, src, dst, b_arr)
    den = den2[:N] + den2[N:]
    dinv = jnp.where(den > 0, 1.0 / den, 0.0)

    gsrc = jnp.concatenate([src, src + N])
    h2 = jnp.concatenate([feat[:, :HD], feat[:, HD:]], axis=0)
    hs2 = [h2]
    for _ in range(K):
        h2 = _hop_stage(h2, gsrc, dst, ee, dinv)
        hs2.append(h2)

    pe = position_emb.reshape(K + 1, D)
    params = jnp.concatenate([
        pe,
        hop_attn_l.reshape(1, D),
        hop_attn_r.reshape(1, D),
        bias.reshape(1, D),
        jnp.zeros((1, D), jnp.float32),
    ], axis=0)
    rst = _combine_stage(hs2, params)
    return rst.reshape(N, 1, D)


# speedup vs baseline: 2.6724x; 1.0566x over previous
import functools

import jax
import jax.numpy as jnp
from jax import lax
from jax.experimental import pallas as pl
from jax.experimental.pallas import tpu as pltpu
from jax.experimental.pallas import tpu_sc as plsc

N = 10000
E = 160000
D = 256
HD = 128
K = 3
NEG = 0.2

CH = 128
NCHUNK = E // CH
NCORE = 2
NSUB = 16
L = 16

_VMESH = plsc.VectorSubcoreMesh(core_axis_name="c", subcore_axis_name="s")

_TSLICES = ((0, 128), (128, 128), (256, 128), (384, 128), (512, 112))

import dataclasses as _dc
_SC_CP = pltpu.CompilerParams()
if "needs_layout_passes" in pltpu.CompilerParams.__dataclass_fields__:
    _SC_CP = _dc.replace(_SC_CP, needs_layout_passes=False)


def _leaky(v):
    return jnp.where(v >= 0, v, NEG * v)


def _fc_body(x_ref, wt_ref, al_ref, ar_ref, feat_ref, el_ref, er_ref):
    x = x_ref[...]
    f = jnp.dot(x, wt_ref[...], preferred_element_type=jnp.float32)
    feat_ref[...] = f
    el_ref[...] = f @ al_ref[...]
    er_ref[...] = f @ ar_ref[...]


def _fc_stage(x, wt, al_col, ar_col):
    B = 2000
    return pl.pallas_call(
        _fc_body,
        grid=(N // B,),
        in_specs=[
            pl.BlockSpec((B, D), lambda i: (i, 0)),
            pl.BlockSpec((D, D), lambda i: (0, 0)),
            pl.BlockSpec((D, 128), lambda i: (0, 0)),
            pl.BlockSpec((D, 128), lambda i: (0, 0)),
        ],
        out_specs=[
            pl.BlockSpec((B, D), lambda i: (i, 0)),
            pl.BlockSpec((B, 128), lambda i: (i, 0)),
            pl.BlockSpec((B, 128), lambda i: (i, 0)),
        ],
        out_shape=[
            jax.ShapeDtypeStruct((N, D), jnp.float32),
            jax.ShapeDtypeStruct((N, 128), jnp.float32),
            jax.ShapeDtypeStruct((N, 128), jnp.float32),
        ],
    )(x, wt, al_col, ar_col)


ECH = 128
EP2 = 163840
ENT = EP2 // ECH // (NCORE * NSUB)
NP2 = 10240


def _edge_body(el_h, er_h, src_h, dst_h, b_h, ee_h, den_h,
               el_v, er_v, b_v, zb_v,
               si0, si1, si2, si3, di0, di1, di2, di3, e0, e1,
               is0, is1, is2, is3, os0, os1, ss0, ss1, den_sh):
    c = lax.axis_index("c")
    s = lax.axis_index("s")
    w = c * NSUB + s

    sis = (si0, si1, si2, si3)
    dis = (di0, di1, di2, di3)
    ebufs = (e0, e1)
    isems = (is0, is1, is2, is3)
    osems = (os0, os1)
    ssems = (ss0, ss1)

    pltpu.sync_copy(el_h, el_v)
    pltpu.sync_copy(er_h, er_v)
    pltpu.sync_copy(b_h, b_v)

    @pl.loop(0, 40)
    def _(i):
        zb_v[pl.ds(i * L, L)] = jnp.zeros((L,), jnp.float32)

    pltpu.sync_copy(zb_v, den_sh.at[pl.ds(s * 640, 640)])

    plsc.subcore_barrier()

    bvec = b_v[...]
    base = w * ENT * ECH

    def issue_idx(j, b):
        pltpu.async_copy(src_h.at[pl.ds(base + j * ECH, ECH)], sis[b], isems[b])
        pltpu.async_copy(dst_h.at[pl.ds(base + j * ECH, ECH)], dis[b], isems[b])

    def wait_idx(j, b):
        pltpu.make_async_copy(src_h.at[pl.ds(base + j * ECH, ECH)], sis[b], isems[b]).wait()
        pltpu.make_async_copy(dst_h.at[pl.ds(base + j * ECH, ECH)], dis[b], isems[b]).wait()

    def compute(b4, b2):
        sv = sis[b4]
        dv = dis[b4]
        ev = ebufs[b2]

        @pl.loop(0, ECH // L)
        def _(g):
            sl = pl.ds(g * L, L)
            e = plsc.load_gather(el_v, [sv[sl]]) + plsc.load_gather(er_v, [dv[sl]])
            e = jnp.where(e >= 0, e, NEG * e)
            ev[sl] = jnp.exp(e - bvec)

    def issue_out(j, b2):
        pltpu.async_copy(ebufs[b2], ee_h.at[pl.ds(base + j * ECH, ECH)], osems[b2])

    def wait_out(j, b2):
        pltpu.make_async_copy(ebufs[b2], ee_h.at[pl.ds(base + j * ECH, ECH)], osems[b2]).wait()

    def issue_scatter(b4, b2):
        pltpu.async_copy(ebufs[b2], den_sh.at[dis[b4]], ssems[b2], add=True)

    def wait_scatter(b4, b2):
        pltpu.make_async_copy(ebufs[b2], den_sh.at[dis[b4]], ssems[b2]).wait()

    def stepe(j, b4, b2, wait_prev, issue_i):
        wait_idx(j, b4)
        if wait_prev:
            wait_out(j - 2, b2)
            wait_scatter((b4 + 2) % 4, b2)
        compute(b4, b2)
        issue_out(j, b2)
        issue_scatter(b4, b2)
        if issue_i:
            issue_idx(j + 2, (b4 + 2) % 4)

    issue_idx(0, 0)
    issue_idx(1, 1)
    issue_idx(2, 2)
    issue_idx(3, 3)
    stepe(0, 0, 0, False, False)
    stepe(1, 1, 1, False, False)

    @pl.loop(0, (ENT - 4) // 4)
    def _(g):
        for r in range(4):
            stepe(2 + g * 4 + r, (2 + r) % 4, r % 2, True, True)

    stepe(ENT - 2, (ENT - 2) % 4, (ENT - 2) % 2, True, False)
    stepe(ENT - 1, (ENT - 1) % 4, (ENT - 1) % 2, True, False)
    wait_out(ENT - 2, (ENT - 2) % 2)
    wait_scatter((ENT - 2) % 4, (ENT - 2) % 2)
    wait_out(ENT - 1, (ENT - 1) % 2)
    wait_scatter((ENT - 1) % 4, (ENT - 1) % 2)

    plsc.subcore_barrier()

    pltpu.sync_copy(den_sh.at[pl.ds(s * 640, 640)], zb_v)
    pltpu.sync_copy(zb_v, den_h.at[pl.ds(c * NP2 + s * 640, 640)])


def _edge_stage(el, er, srcp, dstp, b_arr):
    f = pl.kernel(
        _edge_body,
        out_type=[
            jax.ShapeDtypeStruct((EP2,), jnp.float32),
            jax.ShapeDtypeStruct((NCORE * NP2,), jnp.float32),
        ],
        mesh=_VMESH,
        compiler_params=_SC_CP,
        scratch_types=(
            [pltpu.VMEM((N,), jnp.float32),
             pltpu.VMEM((N,), jnp.float32),
             pltpu.VMEM((L,), jnp.float32),
             pltpu.VMEM((640,), jnp.float32)]
            + [pltpu.VMEM((ECH,), jnp.int32) for _ in range(4)]
            + [pltpu.VMEM((ECH,), jnp.int32) for _ in range(4)]
            + [pltpu.VMEM((ECH,), jnp.float32) for _ in range(2)]
            + [pltpu.SemaphoreType.DMA for _ in range(8)]
            + [pltpu.VMEM_SHARED((NP2,), jnp.float32)]
        ),
    )
    return f(el, er, srcp, dstp, b_arr)


HCH = 80
HNCH = E // HCH
HNT = HNCH // NSUB


def _hop_body(h2_h, gsrc_h, dst_h, a_h, dinv_h, hn2_h,
              gi0, gi1, gi2, gi3, di0, di1, di2, di3, ai0, ai1, ai2, ai3,
              g0, g1, o0, o1, dinv_v,
              is0, is1, is2, is3, gsem0, gsem1, ssem0, ssem1, acc_sh):
    c = lax.axis_index("c")
    s = lax.axis_index("s")

    gis = (gi0, gi1, gi2, gi3)
    dis = (di0, di1, di2, di3)
    ais = (ai0, ai1, ai2, ai3)
    isems = (is0, is1, is2, is3)
    gbufs = (g0, g1)
    obufs = (o0, o1)
    gsems = (gsem0, gsem1)
    ssems = (ssem0, ssem1)

    @pl.loop(0, HCH)
    def _(i):
        row = g0.at[i]
        for j in range(HD // L):
            row[pl.ds(j * L, L)] = jnp.zeros((L,), jnp.float32)

    for i in range(7):
        pltpu.sync_copy(g0.at[pl.ds(0, HCH)],
                        acc_sh.at[pl.ds(s * 624 + i * HCH, HCH)])
    pltpu.sync_copy(g0.at[pl.ds(0, 64)], acc_sh.at[pl.ds(s * 624 + 560, 64)])

    @pl.when(s == NSUB - 1)
    def _():
        pltpu.sync_copy(g0.at[pl.ds(0, 16)], acc_sh.at[pl.ds(9984, 16)])

    pltpu.sync_copy(dinv_h.at[pl.ds(s * 624, 640)], dinv_v)

    plsc.subcore_barrier()

    base_chunk = s * HNT

    def cbase(j):
        return (base_chunk + j) * HCH

    def issue_idx(j, b):
        pltpu.async_copy(gsrc_h.at[pl.ds(c * E + cbase(j), HCH)], gis[b], isems[b])
        pltpu.async_copy(dst_h.at[pl.ds(cbase(j), HCH)], dis[b], isems[b])
        pltpu.async_copy(a_h.at[pl.ds(cbase(j), HCH)], ais[b], isems[b])

    def wait_idx(j, b):
        pltpu.make_async_copy(gsrc_h.at[pl.ds(c * E + cbase(j), HCH)], gis[b], isems[b]).wait()
        pltpu.make_async_copy(dst_h.at[pl.ds(cbase(j), HCH)], dis[b], isems[b]).wait()
        pltpu.make_async_copy(a_h.at[pl.ds(cbase(j), HCH)], ais[b], isems[b]).wait()

    def issue_gather(b4, b2):
        pltpu.async_copy(h2_h.at[gis[b4]], gbufs[b2], gsems[b2])

    def wait_gather(b4, b2):
        pltpu.make_async_copy(h2_h.at[gis[b4]], gbufs[b2], gsems[b2]).wait()

    def do_scale(b4, b2):
        arow = ais[b4]
        g = gbufs[b2]
        o = obufs[b2]

        @plsc.parallel_loop(0, HCH, unroll=8)
        def _(i):
            bc = plsc.load_gather(arow, [jnp.full((L,), i, jnp.int32)])
            rg = g.at[i]
            ro = o.at[i]
            for f in range(HD // L):
                slf = pl.ds(f * L, L)
                ro[slf] = rg[slf] * bc

    def issue_scatter(b4, b2):
        pltpu.async_copy(obufs[b2], acc_sh.at[dis[b4]], ssems[b2], add=True)

    def wait_scatter(b4, b2):
        pltpu.make_async_copy(obufs[b2], acc_sh.at[dis[b4]], ssems[b2]).wait()

    def step(j, b4, b2, wait_sc, issue_i):
        wait_idx(j + 1, (b4 + 1) % 4)
        issue_gather((b4 + 1) % 4, (b2 + 1) % 2)
        wait_gather(b4, b2)
        if wait_sc:
            wait_scatter((b4 + 2) % 4, b2)
        do_scale(b4, b2)
        issue_scatter(b4, b2)
        if issue_i:
            issue_idx(j + 2, (b4 + 2) % 4)

    issue_idx(0, 0)
    issue_idx(1, 1)
    issue_idx(2, 2)
    wait_idx(0, 0)
    issue_gather(0, 0)
    wait_idx(1, 1)
    issue_gather(1, 1)
    wait_gather(0, 0)
    do_scale(0, 0)
    issue_scatter(0, 0)
    wait_idx(2, 2)
    issue_gather(2, 0)
    wait_gather(1, 1)
    do_scale(1, 1)
    issue_scatter(1, 1)
    issue_idx(3, 3)

    @pl.loop(0, (HNT - 5) // 4)
    def _(g):
        for r in range(4):
            step(2 + g * 4 + r, (2 + r) % 4, r % 2, True, True)

    step(HNT - 3, (HNT - 3) % 4, (HNT - 3) % 2, True, True)
    wait_idx(HNT - 1, (HNT - 1) % 4)
    issue_gather((HNT - 1) % 4, (HNT - 1) % 2)
    wait_gather((HNT - 2) % 4, (HNT - 2) % 2)
    wait_scatter((HNT - 4) % 4, (HNT - 4) % 2)
    do_scale((HNT - 2) % 4, (HNT - 2) % 2)
    issue_scatter((HNT - 2) % 4, (HNT - 2) % 2)
    wait_gather((HNT - 1) % 4, (HNT - 1) % 2)
    wait_scatter((HNT - 3) % 4, (HNT - 3) % 2)
    do_scale((HNT - 1) % 4, (HNT - 1) % 2)
    issue_scatter((HNT - 1) % 4, (HNT - 1) % 2)
    wait_scatter((HNT - 2) % 4, (HNT - 2) % 2)
    wait_scatter((HNT - 1) % 4, (HNT - 1) % 2)

    plsc.subcore_barrier()

    def scale_out(local0, nrows):
        @plsc.parallel_loop(0, nrows, unroll=4)
        def _(r):
            bc = plsc.load_gather(dinv_v, [jnp.full((L,), local0 + r, jnp.int32)])
            rg = g0.at[r]
            for f in range(HD // L):
                slf = pl.ds(f * L, L)
                rg[slf] = rg[slf] * bc

    coff = c * N
    for i in range(7):
        row0 = s * 624 + i * HCH
        pltpu.sync_copy(acc_sh.at[pl.ds(row0, HCH)], g0.at[pl.ds(0, HCH)])
        scale_out(i * HCH, HCH)
        pltpu.sync_copy(g0.at[pl.ds(0, HCH)], hn2_h.at[pl.ds(coff + row0, HCH)])
    row0 = s * 624 + 560
    pltpu.sync_copy(acc_sh.at[pl.ds(row0, 64)], g0.at[pl.ds(0, 64)])
    scale_out(560, 64)
    pltpu.sync_copy(g0.at[pl.ds(0, 64)], hn2_h.at[pl.ds(coff + row0, 64)])

    @pl.when(s == NSUB - 1)
    def _():
        pltpu.sync_copy(acc_sh.at[pl.ds(9984, 16)], g0.at[pl.ds(0, 16)])
        scale_out(624, 16)
        pltpu.sync_copy(g0.at[pl.ds(0, 16)], hn2_h.at[pl.ds(coff + 9984, 16)])


def _hop_stage(h2, gsrc, dst, ee, dinv):
    f = pl.kernel(
        _hop_body,
        out_type=jax.ShapeDtypeStruct((NCORE * N, HD), jnp.float32),
        mesh=_VMESH,
        compiler_params=_SC_CP,
        scratch_types=(
            [pltpu.VMEM((HCH,), jnp.int32) for _ in range(4)]
            + [pltpu.VMEM((HCH,), jnp.int32) for _ in range(4)]
            + [pltpu.VMEM((HCH,), jnp.float32) for _ in range(4)]
            + [pltpu.VMEM((HCH, HD), jnp.float32) for _ in range(4)]
            + [pltpu.VMEM((640,), jnp.float32)]
            + [pltpu.SemaphoreType.DMA for _ in range(8)]
            + [pltpu.VMEM_SHARED((N, HD), jnp.float32)]
        ),
    )
    return f(h2, gsrc, dst, ee, dinv)


def _combine_body(h0a, h0b, h1a, h1b, h2a, h2b, h3a, h3b, p_ref, out_ref):
    p = p_ref[...]
    ha = [h0a[...], h1a[...], h2a[...], h3a[...]]
    hb = [h0b[...], h1b[...], h2b[...], h3b[...]]
    for k in range(K + 1):
        ha[k] = ha[k] + p[k:k + 1, :HD]
        hb[k] = hb[k] + p[k:k + 1, HD:]
    hal_a, hal_b = p[4:5, :HD], p[4:5, HD:]
    har_a, har_b = p[5:6, :HD], p[5:6, HD:]
    a_r = jnp.sum(ha[0] * har_a, axis=1, keepdims=True) + \
        jnp.sum(hb[0] * har_b, axis=1, keepdims=True)
    logits = [jnp.sum(ha[k] * hal_a, axis=1, keepdims=True) +
              jnp.sum(hb[k] * hal_b, axis=1, keepdims=True) + a_r
              for k in range(K + 1)]
    logits = [_leaky(lg) for lg in logits]
    mx = functools.reduce(jnp.maximum, logits)
    exps = [jnp.exp(lg - mx) for lg in logits]
    den = functools.reduce(jnp.add, exps)
    outa = functools.reduce(jnp.add, [ha[k] * (exps[k] / den) for k in range(K + 1)])
    outb = functools.reduce(jnp.add, [hb[k] * (exps[k] / den) for k in range(K + 1)])
    out_ref[:, :HD] = outa + p[6:7, :HD]
    out_ref[:, HD:] = outb + p[6:7, HD:]


def _combine_stage(hs2, params):
    B = 2000
    NB = N // B
    ins = []
    specs = []
    for h2 in hs2:
        ins.append(h2)
        specs.append(pl.BlockSpec((B, HD), lambda i: (i, 0)))
        ins.append(h2)
        specs.append(pl.BlockSpec((B, HD), lambda i: (NB + i, 0)))
    ins.append(params)
    specs.append(pl.BlockSpec((8, D), lambda i: (0, 0)))
    return pl.pallas_call(
        _combine_body,
        grid=(NB,),
        in_specs=specs,
        out_specs=pl.BlockSpec((B, D), lambda i: (i, 0)),
        out_shape=jax.ShapeDtypeStruct((N, D), jnp.float32),
    )(*ins)


def kernel(x, edge_index, W_fc, attn_l, attn_r, hop_attn_l, hop_attn_r, position_emb, bias):
    src = edge_index[0]
    dst = edge_index[1]
    al_col = jnp.broadcast_to(attn_l.reshape(D, 1), (D, 128))
    ar_col = jnp.broadcast_to(attn_r.reshape(D, 1), (D, 128))
    feat, el_w, er_w = _fc_stage(x, W_fc.T, al_col, ar_col)
    el = el_w[:, 0]
    er = er_w[:, 0]

    b_const = jnp.max(el) + jnp.max(er)
    b_arr = jnp.full((L,), b_const, jnp.float32)

    srcp = jnp.concatenate([src, jnp.zeros((EP2 - E,), jnp.int32)])
    dstp = jnp.concatenate([dst, jnp.full((EP2 - E,), 10008, jnp.int32)])
    ee, den2 = _edge_stage(el, er, srcp, dstp, b_arr)
    den = den2[:N] + den2[NP2:NP2 + N]
    dinv = jnp.where(den > 0, 1.0 / den, 0.0)

    gsrc = jnp.concatenate([src, src + N])
    h2 = jnp.concatenate([feat[:, :HD], feat[:, HD:]], axis=0)
    hs2 = [h2]
    for _ in range(K):
        h2 = _hop_stage(h2, gsrc, dst, ee, dinv)
        hs2.append(h2)

    pe = position_emb.reshape(K + 1, D)
    params = jnp.concatenate([
        pe,
        hop_attn_l.reshape(1, D),
        hop_attn_r.reshape(1, D),
        bias.reshape(1, D),
        jnp.zeros((1, D), jnp.float32),
    ], axis=0)
    rst = _combine_stage(hs2, params)
    return rst.reshape(N, 1, D)
